# Initial kernel scaffold; baseline (speedup 1.0000x reference)
import jax
import jax.numpy as jnp
from jax.experimental import pallas as pl


def kernel(pos, cell, W_embed, Wr1, br1, Wr2, br2, Wl, Wout, edge_index, atom_types):
    raise NotImplementedError("write your pallas kernel here")


# speedup vs baseline: 28.4048x; 28.4048x over previous
import functools
import math

import jax
import jax.numpy as jnp
from jax import lax
from jax.experimental import pallas as pl
from jax.experimental.pallas import tpu as pltpu
from jax.experimental.pallas import tpu_sc as plsc

RMAX = 5.0
C = 16
NB = 8
HID = 64
CEK = 2000

NC = 2
NS = 16
CH = 80
GK = 5


def _sc_gather(table, idx3d):
    per_w = idx3d.shape[1]
    K = table.shape[1]
    n_grp = per_w // GK
    Eidx = NC * NS * per_w * CH
    mesh = plsc.VectorSubcoreMesh(core_axis_name="c", subcore_axis_name="s")

    @functools.partial(
        pl.kernel,
        out_type=jax.ShapeDtypeStruct((Eidx, K), jnp.float32),
        mesh=mesh,
        compiler_params=pltpu.CompilerParams(use_tc_tiling_on_sc=False),
        scratch_types=[
            pltpu.VMEM((per_w, CH), jnp.int32),
            pltpu.VMEM((GK, CH, K), jnp.float32),
            pltpu.SemaphoreType.DMA,
            pltpu.SemaphoreType.DMA,
        ],
    )
    def k(tab, idxh, out, idx_v, rows_v, lsem, osem):
        w = lax.axis_index("s") * NC + lax.axis_index("c")
        r0 = w * per_w
        pltpu.sync_copy(idxh.at[w], idx_v)

        def grp(g, carry):
            ds = []
            for b in range(GK):
                j = g * GK + b
                ds.append(pltpu.async_copy(tab.at[idx_v.at[j]], rows_v.at[b], lsem))
            for d in ds:
                d.wait()
            os_ = []
            for b in range(GK):
                j = g * GK + b
                os_.append(pltpu.async_copy(
                    rows_v.at[b], out.at[pl.ds((r0 + j) * CH, CH)], osem))
            for d in os_:
                d.wait()
            return carry

        lax.fori_loop(0, n_grp, grp, 0)

    return k(table, idx3d)


def _sc_scatter(vals, idx3d, NT):
    per_w = idx3d.shape[1]
    K = vals.shape[1]
    n_grp = per_w // GK
    NSTR = 10
    NTS = NT // NSTR
    mesh = plsc.VectorSubcoreMesh(core_axis_name="c", subcore_axis_name="s")
    zeros = jnp.zeros((NTS, K), jnp.float32)

    @functools.partial(
        pl.kernel,
        out_type=jax.ShapeDtypeStruct((NC, NT, K), jnp.float32),
        mesh=mesh,
        compiler_params=pltpu.CompilerParams(use_tc_tiling_on_sc=False),
        scratch_types=[
            pltpu.VMEM((per_w, CH), jnp.int32),
            pltpu.VMEM((GK, CH, K), jnp.float32),
            pltpu.VMEM_SHARED((NT, K), jnp.float32),
            pltpu.SemaphoreType.DMA,
            pltpu.SemaphoreType.DMA,
        ],
    )
    def k(vh, idxh, zh, out, idx_v, v_v, acc, lsem, ssem):
        cid = lax.axis_index("c")
        sid = lax.axis_index("s")
        w = sid * NC + cid
        r0 = w * per_w

        @pl.when(sid < NSTR)
        def _():
            pltpu.sync_copy(zh, acc.at[pl.ds(sid * NTS, NTS)])

        pltpu.sync_copy(idxh.at[w], idx_v)
        plsc.subcore_barrier()

        def grp(g, carry):
            ds = []
            for b in range(GK):
                j = g * GK + b
                ds.append(pltpu.async_copy(
                    vh.at[pl.ds((r0 + j) * CH, CH)], v_v.at[b], lsem))
            for d in ds:
                d.wait()
            ss = []
            for b in range(GK):
                j = g * GK + b
                ss.append(pltpu.async_copy(
                    v_v.at[b], acc.at[idx_v.at[j]], ssem, add=True))
            for d in ss:
                d.wait()
            return carry

        lax.fori_loop(0, n_grp, grp, 0)
        plsc.subcore_barrier()

        @pl.when(sid < NSTR)
        def _():
            pltpu.sync_copy(acc.at[pl.ds(sid * NTS, NTS)],
                            out.at[cid, pl.ds(sid * NTS, NTS)])

    return k(vals, idx3d, zeros)


def _full(shape):
    return pl.BlockSpec(shape, lambda *_: tuple(0 for _ in shape))


def _cek(E):
    return CEK if E % CEK == 0 else E


def _radial_w(B, Wr1l, br1l, Wr2l, br2l):
    q1 = jnp.dot(B, Wr1l, preferred_element_type=jnp.float32) + br1l
    s = jax.nn.sigmoid(q1)
    a = q1 * s
    dsilu = s * (1.0 + q1 * (1.0 - s))
    w = jnp.dot(a, Wr2l, preferred_element_type=jnp.float32) + br2l
    return w, dsilu


def _bessel_cut(r, r2):
    n = lax.broadcasted_iota(jnp.int32, (1, NB), 1).astype(jnp.float32) + 1.0
    kk = n * (jnp.pi / RMAX)
    sn = jnp.sin(kk * r)
    pref = math.sqrt(2.0 / RMAX)
    bes = pref * sn / r
    xx = r / RMAX
    x6 = xx ** 6
    env = 1.0 - 28.0 * x6 + 48.0 * x6 * xx - 21.0 * x6 * xx * xx
    cut = env * (xx < 1.0).astype(jnp.float32)
    return bes * cut


def _embed_body(t_ref, we_ref, out_ref):
    NT_ = we_ref.shape[0]
    types = t_ref[...]
    oh = (types == lax.broadcasted_iota(jnp.int32, (types.shape[0], NT_), 1)
          ).astype(jnp.float32)
    out_ref[...] = jnp.dot(oh, we_ref[...], preferred_element_type=jnp.float32)


def _embed(atom_types, W_embed):
    N = atom_types.shape[0]
    return pl.pallas_call(
        _embed_body,
        in_specs=[_full((N, 1)), _full(W_embed.shape)],
        out_specs=_full((N, C)),
        out_shape=jax.ShapeDtypeStruct((N, C), jnp.float32),
    )(atom_types.reshape(N, 1), W_embed)


def _geom_fwd_body(gs_ref, gd_ref, vec_ref, b_ref):
    gs = gs_ref[...]
    gd = gd_ref[...]
    ps = gs[:, 0:3]; pd = gd[:, 0:3]; ecell = gs[:, 3:6]
    vec0 = pd - ps
    shift = (-(vec0 > RMAX).astype(jnp.float32)
             + (vec0 < -RMAX).astype(jnp.float32)) * ecell
    vec = vec0 + shift
    r2 = jnp.sum(vec * vec, axis=-1, keepdims=True) + 1e-12
    r = jnp.sqrt(r2)
    vec_ref[...] = vec
    b_ref[...] = _bessel_cut(r, r2)


def _geom_fwd(gpos, E):
    ce = _cek(E)
    nE = E // ce
    return pl.pallas_call(
        _geom_fwd_body,
        grid=(nE,),
        in_specs=[pl.BlockSpec((ce, C), lambda i: (i, 0)),
                  pl.BlockSpec((ce, C), lambda i: (i + nE, 0))],
        out_specs=[pl.BlockSpec((ce, 3), lambda i: (i, 0)),
                   pl.BlockSpec((ce, NB), lambda i: (i, 0))],
        out_shape=[jax.ShapeDtypeStruct((E, 3), jnp.float32),
                   jax.ShapeDtypeStruct((E, NB), jnp.float32)],
    )(gpos, gpos)


def _edge_fwd_body(b_ref, h_ref, wr1_ref, br1_ref, wr2_ref, br2_ref, m_ref):
    w, _ = _radial_w(b_ref[...], wr1_ref[...], br1_ref[...],
                     wr2_ref[...], br2_ref[...])
    weff = w[:, :C] + w[:, C:]
    m_ref[...] = weff * h_ref[...]


def _edge_fwd(B, hsrc, Wr1l, br1l, Wr2l, br2l, E):
    E2 = hsrc.shape[0]
    ce = _cek(E)
    nE = E // ce
    return pl.pallas_call(
        _edge_fwd_body,
        grid=(E2 // ce,),
        in_specs=[pl.BlockSpec((ce, NB), lambda i: (i % nE, 0)),
                  pl.BlockSpec((ce, C), lambda i: (i, 0)),
                  _full(Wr1l.shape), _full(br1l.shape),
                  _full(Wr2l.shape), _full(br2l.shape)],
        out_specs=pl.BlockSpec((ce, C), lambda i: (i, 0)),
        out_shape=jax.ShapeDtypeStruct((E2, C), jnp.float32),
    )(B, hsrc, Wr1l, br1l, Wr2l, br2l)


def _node_fwd_body(norm, p_ref, h_ref, wl_ref, out_ref):
    agg = (p_ref[0] + p_ref[1]) * norm
    out_ref[...] = h_ref[...] + jnp.dot(agg, wl_ref[...],
                                        preferred_element_type=jnp.float32)


def _node_fwd(parts, h, Wl0, norm):
    N = h.shape[0]
    return pl.pallas_call(
        functools.partial(_node_fwd_body, norm),
        in_specs=[_full((NC, N, C)), _full((N, C)), _full((C, C))],
        out_specs=_full((N, C)),
        out_shape=jax.ShapeDtypeStruct((N, C), jnp.float32),
    )(parts, h, Wl0)


def _energy_body(F, A, h2_ref, wout_ref, tot_ref):
    at = jnp.dot(h2_ref[...], wout_ref[...], preferred_element_type=jnp.float32)
    tot_ref[...] = jnp.concatenate(
        [jnp.sum(at[f * A:(f + 1) * A]).reshape(1, 1) for f in range(F)], axis=0)


def _energy(h2, Wout, F, A):
    N = h2.shape[0]
    return pl.pallas_call(
        functools.partial(_energy_body, F, A),
        in_specs=[_full((N, C)), _full(Wout.shape)],
        out_specs=_full((F, 1)),
        out_shape=jax.ShapeDtypeStruct((F, 1), jnp.float32),
    )(h2, Wout)


def _edge_bwd1_body(norm, b_ref, h_ref, wr1_ref, br1_ref, wr2_ref, br2_ref,
                    wl_ref, wout_ref, ghs_ref, gb_ref):
    Wr1l = wr1_ref[...]; Wr2l = wr2_ref[...]
    w, dsilu = _radial_w(b_ref[...], Wr1l, br1_ref[...], Wr2l, br2_ref[...])
    weff = w[:, :C] + w[:, C:]
    gv = jnp.dot(wl_ref[...], wout_ref[...],
                 preferred_element_type=jnp.float32).reshape(1, C) * norm
    ghs_ref[...] = weff * gv
    Gw1 = gv * h_ref[...]
    G_w = jnp.concatenate([Gw1, Gw1], axis=-1)
    G_q = jnp.dot(G_w, Wr2l.T, preferred_element_type=jnp.float32) * dsilu
    gb_ref[...] = jnp.dot(G_q, Wr1l.T, preferred_element_type=jnp.float32)


def _edge_bwd1(B, hsrc1, Wr1l, br1l, Wr2l, br2l, Wl10, Wout, norm, E):
    E2 = hsrc1.shape[0]
    ce = _cek(E)
    nE = E // ce
    return pl.pallas_call(
        functools.partial(_edge_bwd1_body, norm),
        grid=(E2 // ce,),
        in_specs=[pl.BlockSpec((ce, NB), lambda i: (i % nE, 0)),
                  pl.BlockSpec((ce, C), lambda i: (i, 0)),
                  _full(Wr1l.shape), _full(br1l.shape),
                  _full(Wr2l.shape), _full(br2l.shape),
                  _full((C, C)), _full((C, 1))],
        out_specs=[pl.BlockSpec((ce, C), lambda i: (i, 0)),
                   pl.BlockSpec((ce, NB), lambda i: (i, 0))],
        out_shape=[jax.ShapeDtypeStruct((E2, C), jnp.float32),
                   jax.ShapeDtypeStruct((E2, NB), jnp.float32)],
    )(B, hsrc1, Wr1l, br1l, Wr2l, br2l, Wl10, Wout)


def _node_bwd0_body(norm, p_ref, wl_ref, wout_ref, out_ref):
    G = p_ref[0] + p_ref[1] + wout_ref[...].reshape(1, C)
    out_ref[...] = jnp.dot(G, wl_ref[...].T,
                           preferred_element_type=jnp.float32) * norm


def _node_bwd0(parts, Wl00, Wout, norm):
    N = parts.shape[1]
    return pl.pallas_call(
        functools.partial(_node_bwd0_body, norm),
        in_specs=[_full((NC, N, C)), _full((C, C)), _full((C, 1))],
        out_specs=_full((N, C)),
        out_shape=jax.ShapeDtypeStruct((N, C), jnp.float32),
    )(parts, Wl00, Wout)


def _edge_bwd0_body(b_ref, h_ref, gm_ref, wr1_ref, br1_ref, wr2_ref, br2_ref,
                    gb_ref):
    Wr1l = wr1_ref[...]; Wr2l = wr2_ref[...]
    _, dsilu = _radial_w(b_ref[...], Wr1l, br1_ref[...], Wr2l, br2_ref[...])
    Gw1 = gm_ref[...] * h_ref[...]
    G_w = jnp.concatenate([Gw1, Gw1], axis=-1)
    G_q = jnp.dot(G_w, Wr2l.T, preferred_element_type=jnp.float32) * dsilu
    gb_ref[...] = jnp.dot(G_q, Wr1l.T, preferred_element_type=jnp.float32)


def _edge_bwd0(B, hsrc0, Gm0, Wr1l, br1l, Wr2l, br2l, E):
    E2 = hsrc0.shape[0]
    ce = _cek(E)
    nE = E // ce
    return pl.pallas_call(
        _edge_bwd0_body,
        grid=(E2 // ce,),
        in_specs=[pl.BlockSpec((ce, NB), lambda i: (i % nE, 0)),
                  pl.BlockSpec((ce, C), lambda i: (i, 0)),
                  pl.BlockSpec((ce, C), lambda i: (i, 0)),
                  _full(Wr1l.shape), _full(br1l.shape),
                  _full(Wr2l.shape), _full(br2l.shape)],
        out_specs=pl.BlockSpec((ce, NB), lambda i: (i, 0)),
        out_shape=jax.ShapeDtypeStruct((E2, NB), jnp.float32),
    )(B, hsrc0, Gm0, Wr1l, br1l, Wr2l, br2l)


def _geom_bwd_body(nE, vec_ref, gb0a_ref, gb0b_ref, gb1a_ref, gb1b_ref, d_ref):
    vec = vec_ref[...]
    GB = gb0a_ref[...] + gb0b_ref[...] + gb1a_ref[...] + gb1b_ref[...]
    r2 = jnp.sum(vec * vec, axis=-1, keepdims=True) + 1e-12
    r = jnp.sqrt(r2)
    n = lax.broadcasted_iota(jnp.int32, (1, NB), 1).astype(jnp.float32) + 1.0
    kk = n * (jnp.pi / RMAX)
    sn = jnp.sin(kk * r); cn = jnp.cos(kk * r)
    pref = math.sqrt(2.0 / RMAX)
    bes = pref * sn / r
    dbes = pref * (kk * cn / r - sn / r2)
    xx = r / RMAX
    x5 = xx ** 5
    env = 1.0 - 28.0 * x5 * xx + 48.0 * x5 * xx * xx - 21.0 * x5 * xx ** 3
    denv = (-168.0 * x5 + 336.0 * x5 * xx - 168.0 * x5 * xx * xx) / RMAX
    ind = (xx < 1.0).astype(jnp.float32)
    dB = dbes * env * ind + bes * denv * ind
    Gr = jnp.sum(GB * dB, axis=-1, keepdims=True)
    sign = jnp.where(pl.program_id(0) < nE, 1.0, -1.0)
    D = (sign * Gr / r) * vec
    d_ref[...] = jnp.concatenate(
        [D, jnp.zeros((D.shape[0], C - 3), jnp.float32)], axis=-1)


def _geom_bwd(vecA, GB0, GB1, E):
    E2 = GB0.shape[0]
    ce = _cek(E)
    nE = E // ce
    return pl.pallas_call(
        functools.partial(_geom_bwd_body, nE),
        grid=(E2 // ce,),
        in_specs=[pl.BlockSpec((ce, 3), lambda i: (i % nE, 0)),
                  pl.BlockSpec((ce, NB), lambda i: (i % nE, 0)),
                  pl.BlockSpec((ce, NB), lambda i: (i % nE + nE, 0)),
                  pl.BlockSpec((ce, NB), lambda i: (i % nE, 0)),
                  pl.BlockSpec((ce, NB), lambda i: (i % nE + nE, 0))],
        out_specs=pl.BlockSpec((ce, C), lambda i: (i, 0)),
        out_shape=jax.ShapeDtypeStruct((E2, C), jnp.float32),
    )(vecA, GB0, GB0, GB1, GB1)


@jax.jit
def kernel(pos, cell, W_embed, Wr1, br1, Wr2, br2, Wl, Wout, edge_index, atom_types):
    N = pos.shape[0]
    F = cell.shape[0]
    A = N // F
    E = edge_index.shape[1]
    E2 = 2 * E
    norm = 1.0 / math.sqrt(2.0 * E / float(N))

    src = edge_index[0]; dst = edge_index[1]
    NW = NC * NS
    src2 = jnp.concatenate([src, dst]).reshape(NW, E2 // (NW * CH), CH)
    dst2 = jnp.concatenate([dst, src]).reshape(NW, E2 // (NW * CH), CH)

    repcell = jnp.repeat(cell, A, axis=0)
    ptab = jnp.concatenate(
        [pos, repcell, jnp.zeros((N, C - 6), jnp.float32)], axis=-1)
    h0 = _embed(atom_types.astype(jnp.int32), W_embed)

    br1r = br1.reshape(2, 1, HID); br2r = br2.reshape(2, 1, 2 * C)

    gpos = _sc_gather(ptab, src2)
    vecA, B = _geom_fwd(gpos, E)

    hsrc0 = _sc_gather(h0, src2)
    m0 = _edge_fwd(B, hsrc0, Wr1[0], br1r[0], Wr2[0], br2r[0], E)
    p0 = _sc_scatter(m0, dst2, N)
    h1 = _node_fwd(p0, h0, Wl[0, 0], norm)

    hsrc1 = _sc_gather(h1, src2)
    m1 = _edge_fwd(B, hsrc1, Wr1[1], br1r[1], Wr2[1], br2r[1], E)
    p1 = _sc_scatter(m1, dst2, N)
    h2 = _node_fwd(p1, h1, Wl[1, 0], norm)

    total = _energy(h2, Wout, F, A)[:, 0]

    Ghs1, GB1 = _edge_bwd1(B, hsrc1, Wr1[1], br1r[1], Wr2[1], br2r[1],
                           Wl[1, 0], Wout, norm, E)
    pg = _sc_scatter(Ghs1, src2, N)
    Ghat0 = _node_bwd0(pg, Wl[0, 0], Wout, norm)
    Gm0 = _sc_gather(Ghat0, dst2)
    GB0 = _edge_bwd0(B, hsrc0, Gm0, Wr1[0], br1r[0], Wr2[0], br2r[0], E)

    D2 = _geom_bwd(vecA, GB0, GB1, E)
    pf = _sc_scatter(D2, dst2, N)
    force = -(pf[0, :, 0:3] + pf[1, :, 0:3])
    return total, force.reshape(F, A, 3)


# speedup vs baseline: 29.2167x; 1.0286x over previous
import functools
import math

import jax
import jax.numpy as jnp
from jax import lax
from jax.experimental import pallas as pl
from jax.experimental.pallas import tpu as pltpu
from jax.experimental.pallas import tpu_sc as plsc

RMAX = 5.0
C = 16
NB = 8
HID = 64
CEK = 2000

NC = 2
NS = 16
CH = 80
GK = 25


def _sc_gather(table, idx3d):
    per_w = idx3d.shape[1]
    K = table.shape[1]
    n_grp = per_w // GK
    Eidx = NC * NS * per_w * CH
    mesh = plsc.VectorSubcoreMesh(core_axis_name="c", subcore_axis_name="s")

    @functools.partial(
        pl.kernel,
        out_type=jax.ShapeDtypeStruct((Eidx, K), jnp.float32),
        mesh=mesh,
        compiler_params=pltpu.CompilerParams(use_tc_tiling_on_sc=False),
        scratch_types=[
            pltpu.VMEM((per_w, CH), jnp.int32),
            pltpu.VMEM((GK, CH, K), jnp.float32),
            pltpu.SemaphoreType.DMA,
            pltpu.SemaphoreType.DMA,
        ],
    )
    def k(tab, idxh, out, idx_v, rows_v, lsem, osem):
        w = lax.axis_index("s") * NC + lax.axis_index("c")
        r0 = w * per_w
        pltpu.sync_copy(idxh.at[w], idx_v)

        def grp(g, carry):
            ds = []
            for b in range(GK):
                j = g * GK + b
                ds.append(pltpu.async_copy(tab.at[idx_v.at[j]], rows_v.at[b], lsem))
            for d in ds:
                d.wait()
            os_ = []
            for b in range(GK):
                j = g * GK + b
                os_.append(pltpu.async_copy(
                    rows_v.at[b], out.at[pl.ds((r0 + j) * CH, CH)], osem))
            for d in os_:
                d.wait()
            return carry

        lax.fori_loop(0, n_grp, grp, 0)

    return k(table, idx3d)


def _sc_scatter(vals, idx3d, NT):
    per_w = idx3d.shape[1]
    K = vals.shape[1]
    n_grp = per_w // GK
    NSTR = 10
    NTS = NT // NSTR
    mesh = plsc.VectorSubcoreMesh(core_axis_name="c", subcore_axis_name="s")
    zeros = jnp.zeros((NTS, K), jnp.float32)

    @functools.partial(
        pl.kernel,
        out_type=jax.ShapeDtypeStruct((NC, NT, K), jnp.float32),
        mesh=mesh,
        compiler_params=pltpu.CompilerParams(use_tc_tiling_on_sc=False),
        scratch_types=[
            pltpu.VMEM((per_w, CH), jnp.int32),
            pltpu.VMEM((GK, CH, K), jnp.float32),
            pltpu.VMEM_SHARED((NT, K), jnp.float32),
            pltpu.SemaphoreType.DMA,
            pltpu.SemaphoreType.DMA,
        ],
    )
    def k(vh, idxh, zh, out, idx_v, v_v, acc, lsem, ssem):
        cid = lax.axis_index("c")
        sid = lax.axis_index("s")
        w = sid * NC + cid
        r0 = w * per_w

        @pl.when(sid < NSTR)
        def _():
            pltpu.sync_copy(zh, acc.at[pl.ds(sid * NTS, NTS)])

        pltpu.sync_copy(idxh.at[w], idx_v)
        plsc.subcore_barrier()

        def grp(g, carry):
            ds = []
            for b in range(GK):
                j = g * GK + b
                ds.append(pltpu.async_copy(
                    vh.at[pl.ds((r0 + j) * CH, CH)], v_v.at[b], lsem))
            for d in ds:
                d.wait()
            ss = []
            for b in range(GK):
                j = g * GK + b
                ss.append(pltpu.async_copy(
                    v_v.at[b], acc.at[idx_v.at[j]], ssem, add=True))
            for d in ss:
                d.wait()
            return carry

        lax.fori_loop(0, n_grp, grp, 0)
        plsc.subcore_barrier()

        @pl.when(sid < NSTR)
        def _():
            pltpu.sync_copy(acc.at[pl.ds(sid * NTS, NTS)],
                            out.at[cid, pl.ds(sid * NTS, NTS)])

    return k(vals, idx3d, zeros)


def _full(shape):
    return pl.BlockSpec(shape, lambda *_: tuple(0 for _ in shape))


def _cek(E):
    return CEK if E % CEK == 0 else E


def _radial_w(B, Wr1l, br1l, Wr2l, br2l):
    q1 = jnp.dot(B, Wr1l, preferred_element_type=jnp.float32) + br1l
    s = jax.nn.sigmoid(q1)
    a = q1 * s
    dsilu = s * (1.0 + q1 * (1.0 - s))
    w = jnp.dot(a, Wr2l, preferred_element_type=jnp.float32) + br2l
    return w, dsilu


def _bessel_cut(r, r2):
    n = lax.broadcasted_iota(jnp.int32, (1, NB), 1).astype(jnp.float32) + 1.0
    kk = n * (jnp.pi / RMAX)
    sn = jnp.sin(kk * r)
    pref = math.sqrt(2.0 / RMAX)
    bes = pref * sn / r
    xx = r / RMAX
    x6 = xx ** 6
    env = 1.0 - 28.0 * x6 + 48.0 * x6 * xx - 21.0 * x6 * xx * xx
    cut = env * (xx < 1.0).astype(jnp.float32)
    return bes * cut


def _embed_body(t_ref, we_ref, out_ref):
    NT_ = we_ref.shape[0]
    types = t_ref[...]
    oh = (types == lax.broadcasted_iota(jnp.int32, (types.shape[0], NT_), 1)
          ).astype(jnp.float32)
    out_ref[...] = jnp.dot(oh, we_ref[...], preferred_element_type=jnp.float32)


def _embed(atom_types, W_embed):
    N = atom_types.shape[0]
    return pl.pallas_call(
        _embed_body,
        in_specs=[_full((N, 1)), _full(W_embed.shape)],
        out_specs=_full((N, C)),
        out_shape=jax.ShapeDtypeStruct((N, C), jnp.float32),
    )(atom_types.reshape(N, 1), W_embed)


def _geom_fwd_body(gs_ref, gd_ref, vec_ref, b_ref):
    gs = gs_ref[...]
    gd = gd_ref[...]
    ps = gs[:, 0:3]; pd = gd[:, 0:3]; ecell = gs[:, 3:6]
    vec0 = pd - ps
    shift = (-(vec0 > RMAX).astype(jnp.float32)
             + (vec0 < -RMAX).astype(jnp.float32)) * ecell
    vec = vec0 + shift
    r2 = jnp.sum(vec * vec, axis=-1, keepdims=True) + 1e-12
    r = jnp.sqrt(r2)
    vec_ref[...] = vec
    b_ref[...] = _bessel_cut(r, r2)


def _geom_fwd(gpos, E):
    ce = _cek(E)
    nE = E // ce
    return pl.pallas_call(
        _geom_fwd_body,
        grid=(nE,),
        in_specs=[pl.BlockSpec((ce, 2 * C), lambda i: (i, 0)),
                  pl.BlockSpec((ce, 2 * C), lambda i: (i + nE, 0))],
        out_specs=[pl.BlockSpec((ce, 3), lambda i: (i, 0)),
                   pl.BlockSpec((ce, NB), lambda i: (i, 0))],
        out_shape=[jax.ShapeDtypeStruct((E, 3), jnp.float32),
                   jax.ShapeDtypeStruct((E, NB), jnp.float32)],
    )(gpos, gpos)


def _edge_fwd_body(hoff, b_ref, h_ref, wr1_ref, br1_ref, wr2_ref, br2_ref, m_ref):
    w, _ = _radial_w(b_ref[...], wr1_ref[...], br1_ref[...],
                     wr2_ref[...], br2_ref[...])
    weff = w[:, :C] + w[:, C:]
    m_ref[...] = weff * h_ref[...][:, hoff:hoff + C]


def _edge_fwd(B, hsrc, Wr1l, br1l, Wr2l, br2l, E, hj=0):
    E2 = hsrc.shape[0]
    ce = _cek(E)
    nE = E // ce
    return pl.pallas_call(
        functools.partial(_edge_fwd_body, hj * C),
        grid=(E2 // ce,),
        in_specs=[pl.BlockSpec((ce, NB), lambda i: (i % nE, 0)),
                  pl.BlockSpec((ce, hsrc.shape[1]), lambda i: (i, 0)),
                  _full(Wr1l.shape), _full(br1l.shape),
                  _full(Wr2l.shape), _full(br2l.shape)],
        out_specs=pl.BlockSpec((ce, C), lambda i: (i, 0)),
        out_shape=jax.ShapeDtypeStruct((E2, C), jnp.float32),
    )(B, hsrc, Wr1l, br1l, Wr2l, br2l)


def _node_fwd_body(norm, p_ref, h_ref, wl_ref, out_ref):
    agg = (p_ref[0] + p_ref[1]) * norm
    out_ref[...] = h_ref[...] + jnp.dot(agg, wl_ref[...],
                                        preferred_element_type=jnp.float32)


def _node_fwd(parts, h, Wl0, norm):
    N = h.shape[0]
    return pl.pallas_call(
        functools.partial(_node_fwd_body, norm),
        in_specs=[_full((NC, N, C)), _full((N, C)), _full((C, C))],
        out_specs=_full((N, C)),
        out_shape=jax.ShapeDtypeStruct((N, C), jnp.float32),
    )(parts, h, Wl0)


def _energy_body(F, A, h2_ref, wout_ref, tot_ref):
    at = jnp.dot(h2_ref[...], wout_ref[...], preferred_element_type=jnp.float32)
    tot_ref[...] = jnp.concatenate(
        [jnp.sum(at[f * A:(f + 1) * A]).reshape(1, 1) for f in range(F)], axis=0)


def _energy(h2, Wout, F, A):
    N = h2.shape[0]
    return pl.pallas_call(
        functools.partial(_energy_body, F, A),
        in_specs=[_full((N, C)), _full(Wout.shape)],
        out_specs=_full((F, 1)),
        out_shape=jax.ShapeDtypeStruct((F, 1), jnp.float32),
    )(h2, Wout)


def _edge_bwd1_body(norm, b_ref, h_ref, wr1_ref, br1_ref, wr2_ref, br2_ref,
                    wl_ref, wout_ref, ghs_ref, gb_ref):
    Wr1l = wr1_ref[...]; Wr2l = wr2_ref[...]
    w, dsilu = _radial_w(b_ref[...], Wr1l, br1_ref[...], Wr2l, br2_ref[...])
    weff = w[:, :C] + w[:, C:]
    gv = jnp.dot(wl_ref[...], wout_ref[...],
                 preferred_element_type=jnp.float32).reshape(1, C) * norm
    ghs_ref[...] = weff * gv
    Gw1 = gv * h_ref[...]
    G_w = jnp.concatenate([Gw1, Gw1], axis=-1)
    G_q = jnp.dot(G_w, Wr2l.T, preferred_element_type=jnp.float32) * dsilu
    gb_ref[...] = jnp.dot(G_q, Wr1l.T, preferred_element_type=jnp.float32)


def _edge_bwd1(B, hsrc1, Wr1l, br1l, Wr2l, br2l, Wl10, Wout, norm, E):
    E2 = hsrc1.shape[0]
    ce = _cek(E)
    nE = E // ce
    return pl.pallas_call(
        functools.partial(_edge_bwd1_body, norm),
        grid=(E2 // ce,),
        in_specs=[pl.BlockSpec((ce, NB), lambda i: (i % nE, 0)),
                  pl.BlockSpec((ce, C), lambda i: (i, 0)),
                  _full(Wr1l.shape), _full(br1l.shape),
                  _full(Wr2l.shape), _full(br2l.shape),
                  _full((C, C)), _full((C, 1))],
        out_specs=[pl.BlockSpec((ce, C), lambda i: (i, 0)),
                   pl.BlockSpec((ce, NB), lambda i: (i, 0))],
        out_shape=[jax.ShapeDtypeStruct((E2, C), jnp.float32),
                   jax.ShapeDtypeStruct((E2, NB), jnp.float32)],
    )(B, hsrc1, Wr1l, br1l, Wr2l, br2l, Wl10, Wout)


def _node_bwd0_body(norm, p_ref, wl_ref, wout_ref, out_ref):
    G = p_ref[0] + p_ref[1] + wout_ref[...].reshape(1, C)
    out_ref[...] = jnp.dot(G, wl_ref[...].T,
                           preferred_element_type=jnp.float32) * norm


def _node_bwd0(parts, Wl00, Wout, norm):
    N = parts.shape[1]
    return pl.pallas_call(
        functools.partial(_node_bwd0_body, norm),
        in_specs=[_full((NC, N, C)), _full((C, C)), _full((C, 1))],
        out_specs=_full((N, C)),
        out_shape=jax.ShapeDtypeStruct((N, C), jnp.float32),
    )(parts, Wl00, Wout)


def _edge_bwd0_body(hoff, b_ref, h_ref, gm_ref, wr1_ref, br1_ref, wr2_ref, br2_ref,
                    gb_ref):
    Wr1l = wr1_ref[...]; Wr2l = wr2_ref[...]
    _, dsilu = _radial_w(b_ref[...], Wr1l, br1_ref[...], Wr2l, br2_ref[...])
    Gw1 = gm_ref[...] * h_ref[...][:, hoff:hoff + C]
    G_w = jnp.concatenate([Gw1, Gw1], axis=-1)
    G_q = jnp.dot(G_w, Wr2l.T, preferred_element_type=jnp.float32) * dsilu
    gb_ref[...] = jnp.dot(G_q, Wr1l.T, preferred_element_type=jnp.float32)


def _edge_bwd0(B, hsrc0, Gm0, Wr1l, br1l, Wr2l, br2l, E, hj=0):
    E2 = hsrc0.shape[0]
    ce = _cek(E)
    nE = E // ce
    return pl.pallas_call(
        functools.partial(_edge_bwd0_body, hj * C),
        grid=(E2 // ce,),
        in_specs=[pl.BlockSpec((ce, NB), lambda i: (i % nE, 0)),
                  pl.BlockSpec((ce, hsrc0.shape[1]), lambda i: (i, 0)),
                  pl.BlockSpec((ce, C), lambda i: (i, 0)),
                  _full(Wr1l.shape), _full(br1l.shape),
                  _full(Wr2l.shape), _full(br2l.shape)],
        out_specs=pl.BlockSpec((ce, NB), lambda i: (i, 0)),
        out_shape=jax.ShapeDtypeStruct((E2, NB), jnp.float32),
    )(B, hsrc0, Gm0, Wr1l, br1l, Wr2l, br2l)


def _geom_bwd_body(nE, vec_ref, gb0a_ref, gb0b_ref, gb1a_ref, gb1b_ref, d_ref):
    vec = vec_ref[...]
    GB = gb0a_ref[...] + gb0b_ref[...] + gb1a_ref[...] + gb1b_ref[...]
    r2 = jnp.sum(vec * vec, axis=-1, keepdims=True) + 1e-12
    r = jnp.sqrt(r2)
    n = lax.broadcasted_iota(jnp.int32, (1, NB), 1).astype(jnp.float32) + 1.0
    kk = n * (jnp.pi / RMAX)
    sn = jnp.sin(kk * r); cn = jnp.cos(kk * r)
    pref = math.sqrt(2.0 / RMAX)
    bes = pref * sn / r
    dbes = pref * (kk * cn / r - sn / r2)
    xx = r / RMAX
    x5 = xx ** 5
    env = 1.0 - 28.0 * x5 * xx + 48.0 * x5 * xx * xx - 21.0 * x5 * xx ** 3
    denv = (-168.0 * x5 + 336.0 * x5 * xx - 168.0 * x5 * xx * xx) / RMAX
    ind = (xx < 1.0).astype(jnp.float32)
    dB = dbes * env * ind + bes * denv * ind
    Gr = jnp.sum(GB * dB, axis=-1, keepdims=True)
    sign = jnp.where(pl.program_id(0) < nE, 1.0, -1.0)
    D = (sign * Gr / r) * vec
    d_ref[...] = jnp.concatenate(
        [D, jnp.zeros((D.shape[0], C - 3), jnp.float32)], axis=-1)


def _geom_bwd(vecA, GB0, GB1, E):
    E2 = GB0.shape[0]
    ce = _cek(E)
    nE = E // ce
    return pl.pallas_call(
        functools.partial(_geom_bwd_body, nE),
        grid=(E2 // ce,),
        in_specs=[pl.BlockSpec((ce, 3), lambda i: (i % nE, 0)),
                  pl.BlockSpec((ce, NB), lambda i: (i % nE, 0)),
                  pl.BlockSpec((ce, NB), lambda i: (i % nE + nE, 0)),
                  pl.BlockSpec((ce, NB), lambda i: (i % nE, 0)),
                  pl.BlockSpec((ce, NB), lambda i: (i % nE + nE, 0))],
        out_specs=pl.BlockSpec((ce, C), lambda i: (i, 0)),
        out_shape=jax.ShapeDtypeStruct((E2, C), jnp.float32),
    )(vecA, GB0, GB0, GB1, GB1)


@jax.jit
def kernel(pos, cell, W_embed, Wr1, br1, Wr2, br2, Wl, Wout, edge_index, atom_types):
    N = pos.shape[0]
    F = cell.shape[0]
    A = N // F
    E = edge_index.shape[1]
    E2 = 2 * E
    norm = 1.0 / math.sqrt(2.0 * E / float(N))

    src = edge_index[0]; dst = edge_index[1]
    NW = NC * NS
    src2 = jnp.concatenate([src, dst]).reshape(NW, E2 // (NW * CH), CH)
    dst2 = jnp.concatenate([dst, src]).reshape(NW, E2 // (NW * CH), CH)

    repcell = jnp.repeat(cell, A, axis=0)
    h0 = _embed(atom_types.astype(jnp.int32), W_embed)
    tab0 = jnp.concatenate(
        [pos, repcell, jnp.zeros((N, C - 6), jnp.float32), h0], axis=-1)

    br1r = br1.reshape(2, 1, HID); br2r = br2.reshape(2, 1, 2 * C)

    g0 = _sc_gather(tab0, src2)
    vecA, B = _geom_fwd(g0, E)

    m0 = _edge_fwd(B, g0, Wr1[0], br1r[0], Wr2[0], br2r[0], E, hj=1)
    p0 = _sc_scatter(m0, dst2, N)
    h1 = _node_fwd(p0, h0, Wl[0, 0], norm)

    hsrc1 = _sc_gather(h1, src2)
    m1 = _edge_fwd(B, hsrc1, Wr1[1], br1r[1], Wr2[1], br2r[1], E)
    p1 = _sc_scatter(m1, dst2, N)
    h2 = _node_fwd(p1, h1, Wl[1, 0], norm)

    total = _energy(h2, Wout, F, A)[:, 0]

    Ghs1, GB1 = _edge_bwd1(B, hsrc1, Wr1[1], br1r[1], Wr2[1], br2r[1],
                           Wl[1, 0], Wout, norm, E)
    pg = _sc_scatter(Ghs1, src2, N)
    Ghat0 = _node_bwd0(pg, Wl[0, 0], Wout, norm)
    Gm0 = _sc_gather(Ghat0, dst2)
    GB0 = _edge_bwd0(B, g0, Gm0, Wr1[0], br1r[0], Wr2[0], br2r[0], E, hj=1)

    D2 = _geom_bwd(vecA, GB0, GB1, E)
    pf = _sc_scatter(D2, dst2, N)
    force = -(pf[0, :, 0:3] + pf[1, :, 0:3])
    return total, force.reshape(F, A, 3)
